# Initial kernel scaffold; baseline (speedup 1.0000x reference)
import jax
import jax.numpy as jnp
from jax.experimental import pallas as pl


def kernel(pos, pos_encoder):
    raise NotImplementedError("write your pallas kernel here")


# speedup vs baseline: 2.3229x; 2.3229x over previous
import functools

import jax
import jax.numpy as jnp
from jax import lax
from jax.experimental import pallas as pl
from jax.experimental.pallas import tpu as pltpu
from jax.experimental.pallas import tpu_sc as plsc


@functools.lru_cache(maxsize=None)
def _make_gather(V, D, B):
    info = plsc.get_sparse_core_info()
    NC, NS = info.num_cores, info.num_subcores
    NW = NC * NS
    assert B % NW == 0
    b_per_w = B // NW
    C = 32
    assert b_per_w % C == 0
    n_chunks = b_per_w // C
    mesh = plsc.VectorSubcoreMesh(core_axis_name="c", subcore_axis_name="s")

    @functools.partial(
        pl.kernel,
        mesh=mesh,
        out_type=jax.ShapeDtypeStruct((B, D), jnp.float32),
        scratch_types=[
            pltpu.VMEM((b_per_w,), jnp.int32),
            pltpu.VMEM((2, C, D), jnp.float32),
            pltpu.SemaphoreType.DMA,
            pltpu.SemaphoreType.DMA,
        ],
    )
    def k(table_hbm, idx_hbm, out_hbm, idx_v, rows_v, gsem, osem):
        wid = lax.axis_index("s") * NC + lax.axis_index("c")
        base = wid * b_per_w
        pltpu.sync_copy(idx_hbm.at[pl.ds(base, b_per_w)], idx_v)
        gathers = [
            pltpu.async_copy(
                table_hbm.at[idx_v.at[pl.ds(g * C, C)]], rows_v.at[g % 2], gsem
            )
            for g in range(1)
        ]
        for g in range(n_chunks):
            if g + 1 < n_chunks:
                nxt = pltpu.async_copy(
                    table_hbm.at[idx_v.at[pl.ds((g + 1) * C, C)]],
                    rows_v.at[(g + 1) % 2],
                    gsem,
                )
            gathers[g].wait()
            out = pltpu.async_copy(
                rows_v.at[g % 2], out_hbm.at[pl.ds(base + g * C, C)], osem
            )
            if g + 1 < n_chunks:
                gathers.append(nxt)
            out.wait()

    return k


def kernel(pos, pos_encoder):
    V, D = pos_encoder.shape
    idx = jnp.clip(pos, 0, V - 1).astype(jnp.int32).reshape(-1)
    out = _make_gather(V, D, idx.shape[0])(pos_encoder, idx)
    return out.reshape(pos.shape + (D,))


# speedup vs baseline: 2.3421x; 1.0083x over previous
import functools

import jax
import jax.numpy as jnp
from jax import lax
from jax.experimental import pallas as pl
from jax.experimental.pallas import tpu as pltpu
from jax.experimental.pallas import tpu_sc as plsc


@functools.lru_cache(maxsize=None)
def _make_gather(V, D, B):
    info = plsc.get_sparse_core_info()
    NC, NS = info.num_cores, info.num_subcores
    NW = NC * NS
    assert B % NW == 0
    b_per_w = B // NW
    C = 32
    NBUF = 3
    assert b_per_w % C == 0
    n_chunks = b_per_w // C
    mesh = plsc.VectorSubcoreMesh(core_axis_name="c", subcore_axis_name="s")

    @functools.partial(
        pl.kernel,
        mesh=mesh,
        out_type=jax.ShapeDtypeStruct((B, D), jnp.float32),
        scratch_types=[
            pltpu.VMEM((b_per_w,), jnp.int32),
            pltpu.VMEM((NBUF, C, D), jnp.float32),
            pltpu.SemaphoreType.DMA,
            pltpu.SemaphoreType.DMA,
        ],
    )
    def k(table_hbm, idx_hbm, out_hbm, idx_v, rows_v, gsem, osem):
        wid = lax.axis_index("s") * NC + lax.axis_index("c")
        base = wid * b_per_w
        pltpu.sync_copy(idx_hbm.at[pl.ds(base, b_per_w)], idx_v)

        def gather(g):
            return pltpu.async_copy(
                table_hbm.at[idx_v.at[pl.ds(g * C, C)]], rows_v.at[g % NBUF], gsem
            )

        gathers = [gather(g) for g in range(NBUF - 1)]
        outs = []
        for g in range(n_chunks):
            f = g + NBUF - 1
            if f < n_chunks:
                if f - NBUF >= 0:
                    outs[f - NBUF].wait()
                gathers.append(gather(f))
            gathers[g].wait()
            outs.append(
                pltpu.async_copy(
                    rows_v.at[g % NBUF], out_hbm.at[pl.ds(base + g * C, C)], osem
                )
            )
        for g in range(max(0, n_chunks - NBUF), n_chunks):
            outs[g].wait()

    return k


def kernel(pos, pos_encoder):
    V, D = pos_encoder.shape
    idx = jnp.clip(pos, 0, V - 1).astype(jnp.int32).reshape(-1)
    out = _make_gather(V, D, idx.shape[0])(pos_encoder, idx)
    return out.reshape(pos.shape + (D,))
